# Initial kernel scaffold; baseline (speedup 1.0000x reference)
import jax
import jax.numpy as jnp
from jax.experimental import pallas as pl


def kernel(train_text, train_audio, edge_index, W_rnn_t, b_rnn_t, W_rnn_a, b_rnn_a, a_src_t, a_dst_t, a_src_a, a_dst_a, W1_t, W2_t, W1_a, W2_a, Wc1, bc1, Wc2, bc2):
    raise NotImplementedError("write your pallas kernel here")


# speedup vs baseline: 14.4235x; 14.4235x over previous
import functools

import jax
import jax.numpy as jnp
from jax import lax
from jax.experimental import pallas as pl
from jax.experimental.pallas import tpu as pltpu
from jax.experimental.pallas import tpu_sc as plsc

N = 10000
NP = 10240
E = 320000
NC = 2
NS = 16
NW = NC * NS
EW = E // NW
K = 80
NCH = EW // K
RW = NP // NS
BR = 256
NB = NP // BR
F32 = jnp.float32

_mesh = plsc.VectorSubcoreMesh(core_axis_name="c", subcore_axis_name="s")


def _hi_dot(a, b):
  return lax.dot_general(a, b, (((1,), (0,)), ((), ())),
                         preferred_element_type=F32,
                         precision=lax.Precision.HIGHEST)


def _encode(x, wr, b, a_src, a_dst, w1):
  u = x.shape[1]
  h1 = w1.shape[1]
  a2 = jnp.stack([a_src, a_dst], axis=1)
  b2 = b.reshape(1, u)

  def body(x_ref, wr_ref, b_ref, a2_ref, w1_ref, h_ref, x1_ref, sd_ref):
    h = jnp.tanh(_hi_dot(x_ref[...], wr_ref[...]) + b_ref[...])
    h_ref[...] = h
    x1_ref[...] = _hi_dot(h, w1_ref[...])
    sd_ref[...] = _hi_dot(h, a2_ref[...])

  h, x1, sd = pl.pallas_call(
      body,
      grid=(NB,),
      in_specs=[
          pl.BlockSpec((BR, u), lambda i: (i, 0)),
          pl.BlockSpec((u, u), lambda i: (0, 0)),
          pl.BlockSpec((1, u), lambda i: (0, 0)),
          pl.BlockSpec((u, 2), lambda i: (0, 0)),
          pl.BlockSpec((u, h1), lambda i: (0, 0)),
      ],
      out_specs=[
          pl.BlockSpec((BR, u), lambda i: (i, 0)),
          pl.BlockSpec((BR, h1), lambda i: (i, 0)),
          pl.BlockSpec((BR, 2), lambda i: (i, 0)),
      ],
      out_shape=[
          jax.ShapeDtypeStruct((NP, u), F32),
          jax.ShapeDtypeStruct((NP, h1), F32),
          jax.ShapeDtypeStruct((NP, 2), F32),
      ],
  )(x, wr, b2, a2, w1)
  return h, x1, sd


@functools.partial(
    pl.kernel,
    out_type=[
        jax.ShapeDtypeStruct((NW, EW), F32),
        jax.ShapeDtypeStruct((NC, NP), F32),
    ],
    mesh=_mesh,
    scratch_types=[
        pltpu.VMEM((NP,), F32),
        pltpu.VMEM((NP,), F32),
        pltpu.VMEM((NCH, K), jnp.int32),
        pltpu.VMEM((NCH, K), jnp.int32),
        pltpu.VMEM((EW,), F32),
        pltpu.VMEM((RW,), F32),
        pltpu.VMEM((128,), F32),
        pltpu.VMEM_SHARED((NP,), F32),
    ],
    compiler_params=pltpu.CompilerParams(needs_layout_passes=False, use_tc_tiling_on_sc=False),
)
def _attn_kernel(s_hbm, d_hbm, src_hbm, dst_hbm, ex_out, z_out,
                 sv, dv, srcv, dstv, exbuf, zslice, tmp16, zsh):
  cid = lax.axis_index("c")
  sid = lax.axis_index("s")
  wid = sid * NC + cid
  pltpu.sync_copy(s_hbm, sv)
  pltpu.sync_copy(d_hbm, dv)
  pltpu.sync_copy(src_hbm.at[wid], srcv)
  pltpu.sync_copy(dst_hbm.at[wid], dstv)

  zero16 = jnp.zeros((16,), F32)
  for i in range(RW // 16):
    zslice[pl.ds(i * 16, 16)] = zero16
  pltpu.sync_copy(zslice, zsh.at[pl.ds(sid * RW, RW)])
  plsc.subcore_barrier()

  def maxbody(i, carry):
    ms, md = carry
    return (jnp.maximum(ms, sv[pl.ds(i * 16, 16)]),
            jnp.maximum(md, dv[pl.ds(i * 16, 16)]))

  ms, md = lax.fori_loop(0, NP // 16, maxbody,
                         (jnp.full((16,), -1e30, F32),
                          jnp.full((16,), -1e30, F32)))
  lanes = lax.iota(jnp.int32, 16)

  def lane_max(v):
    for shift in (8, 4, 2, 1):
      tmp16[pl.ds(0, 16)] = v
      v = jnp.maximum(v, plsc.load_gather(tmp16, [(lanes + shift) & 15]))
    return v

  csplat = jnp.maximum(lane_max(ms) + lane_max(md), 0.0)

  def chunk(c, carry):
    for j in range(K // 16):
      si = srcv[c, pl.ds(j * 16, 16)]
      di = dstv[c, pl.ds(j * 16, 16)]
      e = plsc.load_gather(sv, [si]) + plsc.load_gather(dv, [di])
      e = jnp.where(e >= 0.0, e, 0.2 * e)
      exbuf[pl.ds(c * K + j * 16, 16)] = jnp.exp(e - csplat)
    pltpu.sync_copy(exbuf.at[pl.ds(c * K, K)], zsh.at[dstv.at[c]], add=True)
    return carry

  lax.fori_loop(0, NCH, chunk, 0)
  pltpu.sync_copy(exbuf, ex_out.at[wid])
  plsc.subcore_barrier()
  pltpu.sync_copy(zsh.at[pl.ds(sid * RW, RW)],
                  z_out.at[cid, pl.ds(sid * RW, RW)])


def _make_msgpass(d):
  @functools.partial(
      pl.kernel,
      out_type=jax.ShapeDtypeStruct((NC, NP, d), F32),
      mesh=_mesh,
      scratch_types=[
          pltpu.VMEM((NCH, K), jnp.int32),
          pltpu.VMEM((NCH, K), jnp.int32),
          pltpu.VMEM((NCH, K), F32),
          pltpu.VMEM((K, d), F32),
          pltpu.VMEM((16, d), F32),
          pltpu.VMEM_SHARED((NP, d), F32),
          pltpu.SemaphoreType.DMA,
      ],
      compiler_params=pltpu.CompilerParams(needs_layout_passes=False, use_tc_tiling_on_sc=False),
  )
  def msg_kernel(x_hbm, ex_hbm, src_hbm, dst_hbm, acc_out,
                 srcv, dstv, exv, rows, zrow, acc, sem):
    cid = lax.axis_index("c")
    sid = lax.axis_index("s")
    wid = sid * NC + cid
    pltpu.sync_copy(src_hbm.at[wid], srcv)
    pltpu.sync_copy(dst_hbm.at[wid], dstv)
    pltpu.sync_copy(ex_hbm.at[wid], exv)

    zero16 = jnp.zeros((16,), F32)
    for i in range(16):
      for j in range(d // 16):
        zrow[i, pl.ds(j * 16, 16)] = zero16
    for i in range(RW // 16):
      pltpu.sync_copy(zrow, acc.at[pl.ds(sid * RW + i * 16, 16)])
    plsc.subcore_barrier()

    def chunk(c, carry):
      pltpu.async_copy(x_hbm.at[srcv.at[c]], rows, sem).wait()
      cc = jnp.full((16,), c, jnp.int32)
      for k in range(K):
        w = plsc.load_gather(exv, [cc, jnp.full((16,), k, jnp.int32)])
        for j in range(d // 16):
          rows[k, pl.ds(j * 16, 16)] = rows[k, pl.ds(j * 16, 16)] * w
      pltpu.sync_copy(rows, acc.at[dstv.at[c]], add=True)
      return carry

    lax.fori_loop(0, NCH, chunk, 0)
    plsc.subcore_barrier()
    pltpu.sync_copy(acc.at[pl.ds(sid * RW, RW)],
                    acc_out.at[cid, pl.ds(sid * RW, RW)])

  return msg_kernel


_msgpass_128 = _make_msgpass(128)
_msgpass_64 = _make_msgpass(64)


def _layer1(acc, z3, w2):
  h1, h2 = w2.shape

  def body(acc_ref, z_ref, w2_ref, x2_ref):
    den = z_ref[0] + z_ref[1] + 1e-16
    g = jnp.maximum((acc_ref[0] + acc_ref[1]) / den, 0.0)
    x2_ref[...] = _hi_dot(g, w2_ref[...])

  return pl.pallas_call(
      body,
      grid=(NB,),
      in_specs=[
          pl.BlockSpec((NC, BR, h1), lambda i: (0, i, 0)),
          pl.BlockSpec((NC, BR, 1), lambda i: (0, i, 0)),
          pl.BlockSpec((h1, h2), lambda i: (0, 0)),
      ],
      out_specs=pl.BlockSpec((BR, h2), lambda i: (i, 0)),
      out_shape=jax.ShapeDtypeStruct((NP, h2), F32),
  )(acc, z3, w2)


def _final(acc2_t, acc2_a, z3_t, z3_a, h_t, h_a, wc1, bc1, wc2, bc2):
  h2 = acc2_t.shape[2]
  ut = h_t.shape[1]
  ua = h_a.shape[1]
  hc = wc1.shape[1]
  tags = wc2.shape[1]
  w_g2t = wc1[0:h2]
  w_g2a = wc1[h2:2 * h2]
  w_ha = wc1[2 * h2:2 * h2 + ua]
  w_ht = wc1[2 * h2 + ua:]
  bc1r = bc1.reshape(1, hc)
  bc2r = bc2.reshape(1, tags)

  def body(a2t_ref, a2a_ref, zt_ref, za_ref, ht_ref, ha_ref,
           wg2t_ref, wg2a_ref, wha_ref, wht_ref, b1_ref, wc2_ref, b2_ref,
           out_ref):
    g2t = (a2t_ref[0] + a2t_ref[1]) / (zt_ref[0] + zt_ref[1] + 1e-16)
    g2a = (a2a_ref[0] + a2a_ref[1]) / (za_ref[0] + za_ref[1] + 1e-16)
    hid = (_hi_dot(g2t, wg2t_ref[...]) + _hi_dot(g2a, wg2a_ref[...])
           + _hi_dot(ha_ref[...], wha_ref[...])
           + _hi_dot(ht_ref[...], wht_ref[...]) + b1_ref[...])
    hid = jnp.maximum(hid, 0.0)
    lg = _hi_dot(hid, wc2_ref[...]) + b2_ref[...]
    m = jnp.max(lg, axis=1, keepdims=True)
    p = lg - m
    out_ref[...] = p - jnp.log(jnp.sum(jnp.exp(p), axis=1, keepdims=True))

  return pl.pallas_call(
      body,
      grid=(NB,),
      in_specs=[
          pl.BlockSpec((NC, BR, h2), lambda i: (0, i, 0)),
          pl.BlockSpec((NC, BR, h2), lambda i: (0, i, 0)),
          pl.BlockSpec((NC, BR, 1), lambda i: (0, i, 0)),
          pl.BlockSpec((NC, BR, 1), lambda i: (0, i, 0)),
          pl.BlockSpec((BR, ut), lambda i: (i, 0)),
          pl.BlockSpec((BR, ua), lambda i: (i, 0)),
          pl.BlockSpec((h2, hc), lambda i: (0, 0)),
          pl.BlockSpec((h2, hc), lambda i: (0, 0)),
          pl.BlockSpec((ua, hc), lambda i: (0, 0)),
          pl.BlockSpec((ut, hc), lambda i: (0, 0)),
          pl.BlockSpec((1, hc), lambda i: (0, 0)),
          pl.BlockSpec((hc, tags), lambda i: (0, 0)),
          pl.BlockSpec((1, tags), lambda i: (0, 0)),
      ],
      out_specs=pl.BlockSpec((BR, tags), lambda i: (i, 0)),
      out_shape=jax.ShapeDtypeStruct((N, tags), F32),
  )(acc2_t, acc2_a, z3_t, z3_a, h_t, h_a,
    w_g2t, w_g2a, w_ha, w_ht, bc1r, wc2, bc2r)


def kernel(train_text, train_audio, edge_index, W_rnn_t, b_rnn_t, W_rnn_a,
           b_rnn_a, a_src_t, a_dst_t, a_src_a, a_dst_a, W1_t, W2_t, W1_a,
           W2_a, Wc1, bc1, Wc2, bc2):
  src3 = edge_index[0].reshape(NW, NCH, K)
  dst3 = edge_index[1].reshape(NW, NCH, K)
  xt = jnp.pad(train_text, ((0, NP - N), (0, 0)))
  xa = jnp.pad(train_audio, ((0, NP - N), (0, 0)))

  h_t, x1_t, sd_t = _encode(xt, W_rnn_t, b_rnn_t, a_src_t, a_dst_t, W1_t)
  h_a, x1_a, sd_a = _encode(xa, W_rnn_a, b_rnn_a, a_src_a, a_dst_a, W1_a)

  ex_t, z_t = _attn_kernel(sd_t[:, 0], sd_t[:, 1], src3, dst3)
  ex_a, z_a = _attn_kernel(sd_a[:, 0], sd_a[:, 1], src3, dst3)
  ex_t3 = ex_t.reshape(NW, NCH, K)
  ex_a3 = ex_a.reshape(NW, NCH, K)
  z3_t = z_t.reshape(NC, NP, 1)
  z3_a = z_a.reshape(NC, NP, 1)

  acc1_t = _msgpass_128(x1_t, ex_t3, src3, dst3)
  acc1_a = _msgpass_128(x1_a, ex_a3, src3, dst3)

  x2_t = _layer1(acc1_t, z3_t, W2_t)
  x2_a = _layer1(acc1_a, z3_a, W2_a)

  acc2_t = _msgpass_64(x2_t, ex_t3, src3, dst3)
  acc2_a = _msgpass_64(x2_a, ex_a3, src3, dst3)

  return _final(acc2_t, acc2_a, z3_t, z3_a, h_t, h_a, Wc1, bc1, Wc2, bc2)
